# baseline (speedup 1.0000x reference)
import functools

import jax
import jax.numpy as jnp
from jax import lax
from jax.experimental import pallas as pl
from jax.experimental.pallas import tpu as pltpu
from jax.experimental.pallas import tpu_sc as plsc

_BASE_LOD = 9
_N = 262144
_V = 2000000
_D = 16
_L = 16

_PB = 128
_NB = _N // _PB
_RPB = _PB * 8


@functools.lru_cache(maxsize=None)
def _make_kernel(nc, ns):
    nw = nc * ns
    blocks_per_w = _NB // nw
    mesh = plsc.VectorSubcoreMesh(core_axis_name="c", subcore_axis_name="s")

    @functools.partial(
        pl.kernel,
        mesh=mesh,
        compiler_params=pltpu.CompilerParams(
            needs_layout_passes=False, use_tc_tiling_on_sc=False),
        out_type=jax.ShapeDtypeStruct((2, _NB, 8, _PB), jnp.float32),
        scratch_types=[
            pltpu.VMEM((8, _PB), jnp.int32),
            pltpu.VMEM((_RPB, _D), jnp.float32),
            pltpu.VMEM((_PB,), jnp.float32),
            pltpu.VMEM((_PB,), jnp.float32),
            pltpu.VMEM((_PB,), jnp.float32),
            pltpu.VMEM((_D, _PB), jnp.float32),
            pltpu.VMEM((_L,), jnp.float32),
            pltpu.SemaphoreType.DMA,
        ],
    )
    def spc_kernel(x0_hbm, x1_hbm, x2_hbm, cidx_hbm, feat_hbm, res_hbm,
                   out_hbm, idx_v, rows_v, x0_v, x1_v, x2_v, out_v, res_v,
                   sem):
        wid = lax.axis_index("s") * nc + lax.axis_index("c")
        pltpu.sync_copy(res_hbm, res_v)
        res = res_v[...]
        lane = lax.iota(jnp.int32, _L)

        def block_body(b, carry):
            cb = wid * blocks_per_w + b
            pbase = pl.multiple_of(cb * _PB, _PB)
            pltpu.sync_copy(cidx_hbm.at[cb], idx_v)
            pltpu.sync_copy(x0_hbm.at[pl.ds(pbase, _PB)], x0_v)
            pltpu.sync_copy(x1_hbm.at[pl.ds(pbase, _PB)], x1_v)
            pltpu.sync_copy(x2_hbm.at[pl.ds(pbase, _PB)], x2_v)
            copies = [
                pltpu.async_copy(
                    feat_hbm.at[idx_v.at[j]],
                    rows_v.at[pl.ds(j * _PB, _PB)],
                    sem,
                )
                for j in range(8)
            ]
            for cp in copies:
                cp.wait()

            def grp_body(g, carry2):
                s = g * _L
                f0 = x0_v[pl.ds(s, _L)] * res
                f1 = x1_v[pl.ds(s, _L)] * res
                f2 = x2_v[pl.ds(s, _L)] * res
                f0 = f0 - f0.astype(jnp.int32).astype(jnp.float32)
                f1 = f1 - f1.astype(jnp.int32).astype(jnp.float32)
                f2 = f2 - f2.astype(jnp.int32).astype(jnp.float32)
                a0 = 1.0 - f0
                a1 = 1.0 - f1
                a2 = 1.0 - f2
                p00 = a0 * a1
                p01 = a0 * f1
                p10 = f0 * a1
                p11 = f0 * f1
                cs = (p00 * a2, p00 * f2, p01 * a2, p01 * f2,
                      p10 * a2, p10 * f2, p11 * a2, p11 * f2)
                base = s + lane
                ridx = [j * _PB + base for j in range(8)]
                for d in range(_D):
                    cold = jnp.full((_L,), d, jnp.int32)
                    acc = cs[0] * plsc.load_gather(rows_v, [ridx[0], cold])
                    for j in range(1, 8):
                        acc = acc + cs[j] * plsc.load_gather(
                            rows_v, [ridx[j], cold])
                    out_v[d, pl.ds(s, _L)] = acc
                return carry2

            lax.fori_loop(0, _PB // _L, grp_body, 0)
            pltpu.sync_copy(out_v.at[pl.ds(0, 8)], out_hbm.at[0, cb])
            pltpu.sync_copy(out_v.at[pl.ds(8, 8)], out_hbm.at[1, cb])
            return carry

        lax.fori_loop(0, blocks_per_w, block_body, 0)

    return spc_kernel


def kernel(x, corner_idx, features, lod):
    res = (jnp.asarray(2, jnp.int32) ** (lod + _BASE_LOD)).astype(jnp.float32)
    res_vec = jnp.full((_L,), 1.0, jnp.float32) * res
    x0 = x[:, 0]
    x1 = x[:, 1]
    x2 = x[:, 2]
    cidx_view = corner_idx.T.reshape(8, _NB, _PB).transpose(1, 0, 2)
    info = plsc.get_sparse_core_info()
    k = _make_kernel(info.num_cores, info.num_subcores)
    out_view = k(x0, x1, x2, cidx_view, features, res_vec)
    return out_view.transpose(0, 2, 1, 3).reshape(_D, _N).T


# speedup vs baseline: 1.3058x; 1.3058x over previous
import functools

import jax
import jax.numpy as jnp
from jax import lax
from jax.experimental import pallas as pl
from jax.experimental.pallas import tpu as pltpu
from jax.experimental.pallas import tpu_sc as plsc

_BASE_LOD = 9
_N = 262144
_V = 2000000
_D = 16
_L = 16

_PB = 128
_NB = _N // _PB
_RPB = _PB * 8


@functools.lru_cache(maxsize=None)
def _make_kernel(nc, ns):
    nw = nc * ns
    bpw = _NB // nw
    ppw = _N // nw
    mesh = plsc.VectorSubcoreMesh(core_axis_name="c", subcore_axis_name="s")

    @functools.partial(
        pl.kernel,
        mesh=mesh,
        compiler_params=pltpu.CompilerParams(
            needs_layout_passes=False, use_tc_tiling_on_sc=False),
        out_type=jax.ShapeDtypeStruct((_N, _D), jnp.float32),
        scratch_types=[
            pltpu.VMEM((2, 8, _PB), jnp.int32),
            pltpu.VMEM((2, _RPB, _D), jnp.float32),
            pltpu.VMEM((ppw,), jnp.float32),
            pltpu.VMEM((ppw,), jnp.float32),
            pltpu.VMEM((ppw,), jnp.float32),
            pltpu.VMEM((_RPB,), jnp.float32),
            pltpu.VMEM((_PB, _D), jnp.float32),
            pltpu.VMEM((_L,), jnp.float32),
            pltpu.SemaphoreType.DMA((2,)),
        ],
    )
    def spc_kernel(x0_hbm, x1_hbm, x2_hbm, cidx_hbm, feat_hbm, res_hbm,
                   out_hbm, idx_v, rows_v, x0_v, x1_v, x2_v, coeff_v, out_v,
                   res_v, sems):
        wid = lax.axis_index("s") * nc + lax.axis_index("c")
        base_blk = wid * bpw
        base_pt = wid * ppw
        pltpu.sync_copy(res_hbm, res_v)
        pltpu.sync_copy(x0_hbm.at[pl.ds(pl.multiple_of(base_pt, ppw), ppw)],
                        x0_v)
        pltpu.sync_copy(x1_hbm.at[pl.ds(pl.multiple_of(base_pt, ppw), ppw)],
                        x1_v)
        pltpu.sync_copy(x2_hbm.at[pl.ds(pl.multiple_of(base_pt, ppw), ppw)],
                        x2_v)
        res = res_v[...]
        lane = lax.iota(jnp.int32, _L)

        def issue(blk, q):
            pltpu.sync_copy(cidx_hbm.at[blk], idx_v.at[q])
            for j in range(8):
                pltpu.async_copy(
                    feat_hbm.at[idx_v.at[q, j]],
                    rows_v.at[q, pl.ds(j * _PB, _PB)],
                    sems.at[q],
                )

        issue(base_blk, 0)

        def block_body(b, carry):
            p = jnp.bitwise_and(b, 1)
            q = 1 - p

            @pl.when(b < bpw - 1)
            def _():
                issue(base_blk + b + 1, q)

            pltpu.make_async_copy(
                feat_hbm.at[pl.ds(0, _RPB)], rows_v.at[p], sems.at[p]
            ).wait()

            s_in_w = b * _PB

            def grp_body(g, carry2):
                s = g * _L
                f0 = x0_v[pl.ds(s_in_w + s, _L)] * res
                f1 = x1_v[pl.ds(s_in_w + s, _L)] * res
                f2 = x2_v[pl.ds(s_in_w + s, _L)] * res
                f0 = f0 - f0.astype(jnp.int32).astype(jnp.float32)
                f1 = f1 - f1.astype(jnp.int32).astype(jnp.float32)
                f2 = f2 - f2.astype(jnp.int32).astype(jnp.float32)
                a0 = 1.0 - f0
                a1 = 1.0 - f1
                a2 = 1.0 - f2
                p00 = a0 * a1
                p01 = a0 * f1
                p10 = f0 * a1
                p11 = f0 * f1
                cs = (p00 * a2, p00 * f2, p01 * a2, p01 * f2,
                      p10 * a2, p10 * f2, p11 * a2, p11 * f2)
                sbase = lane * 8 + s * 8
                for j in range(8):
                    plsc.store_scatter(coeff_v, [sbase + j], cs[j])
                return carry2

            lax.fori_loop(0, _PB // _L, grp_body, 0)

            def pair_body(m, carry2):
                k = m * 2
                r = m * 16
                cv = coeff_v[pl.ds(r, _L)]
                acc0 = cv[0] * rows_v[p, k]
                acc1 = cv[8] * rows_v[p, k + 1]
                for j in range(1, 8):
                    acc0 = acc0 + cv[j] * rows_v[p, j * _PB + k]
                    acc1 = acc1 + cv[8 + j] * rows_v[p, j * _PB + k + 1]
                out_v[k] = acc0
                out_v[k + 1] = acc1
                return carry2

            lax.fori_loop(0, _PB // 2, pair_body, 0)
            pbase = pl.multiple_of((base_blk + b) * _PB, _PB)
            pltpu.sync_copy(out_v, out_hbm.at[pl.ds(pbase, _PB)])
            return carry

        lax.fori_loop(0, bpw, block_body, 0)

    return spc_kernel


def kernel(x, corner_idx, features, lod):
    res = (jnp.asarray(2, jnp.int32) ** (lod + _BASE_LOD)).astype(jnp.float32)
    res_vec = jnp.full((_L,), 1.0, jnp.float32) * res
    x0 = x[:, 0]
    x1 = x[:, 1]
    x2 = x[:, 2]
    cidx_view = corner_idx.T.reshape(8, _NB, _PB).transpose(1, 0, 2)
    info = plsc.get_sparse_core_info()
    k = _make_kernel(info.num_cores, info.num_subcores)
    return k(x0, x1, x2, cidx_view, features, res_vec)


# speedup vs baseline: 2.3439x; 1.7951x over previous
import functools

import jax
import jax.numpy as jnp
from jax import lax
from jax.experimental import pallas as pl
from jax.experimental.pallas import tpu as pltpu
from jax.experimental.pallas import tpu_sc as plsc

_BASE_LOD = 9
_N = 262144
_V = 2000000
_D = 16
_L = 16

_PB = 128
_NB = _N // _PB
_RPB = _PB * 8


@functools.lru_cache(maxsize=None)
def _make_convert(nc, ns):
    nw = nc * ns
    nchunk = _V // 128
    per_w = (nchunk + nw - 1) // nw
    mesh = plsc.VectorSubcoreMesh(core_axis_name="c", subcore_axis_name="s")

    @functools.partial(
        pl.kernel,
        mesh=mesh,
        compiler_params=pltpu.CompilerParams(
            needs_layout_passes=False, use_tc_tiling_on_sc=False),
        out_type=jax.ShapeDtypeStruct((_V, _D), jnp.float32),
        scratch_types=[
            pltpu.VMEM((2, _D, 129), jnp.float32),
            pltpu.VMEM((2, 128, _D), jnp.float32),
            pltpu.SemaphoreType.DMA((2,)),
            pltpu.SemaphoreType.DMA((2,)),
        ],
    )
    def conv_kernel(fv_hbm, out_hbm, pad_v, out_v, isems, osems):
        wid = lax.axis_index("s") * nc + lax.axis_index("c")
        start = wid * per_w
        cnt = jnp.minimum(per_w, nchunk - start)
        lane = lax.iota(jnp.int32, _L)

        def issue(c, q):
            m = c * 8
            pltpu.async_copy(fv_hbm.at[0, pl.ds(m, 8)],
                             pad_v.at[q, pl.ds(0, 8), pl.ds(0, 128)],
                             isems.at[q])
            pltpu.async_copy(fv_hbm.at[1, pl.ds(m, 8)],
                             pad_v.at[q, pl.ds(8, 8), pl.ds(0, 128)],
                             isems.at[q])

        issue(start, 0)

        def body(i, carry):
            p = jnp.bitwise_and(i, 1)
            q = 1 - p
            c = start + i

            @pl.when(i + 1 < cnt)
            def _():
                issue(c + 1, q)

            pltpu.make_async_copy(
                fv_hbm.at[0, pl.ds(0, 16)],
                pad_v.at[p, pl.ds(0, 16), pl.ds(0, 128)],
                isems.at[p],
            ).wait()

            @pl.when(i >= 2)
            def _():
                pltpu.make_async_copy(
                    out_v.at[p], out_hbm.at[pl.ds(0, 128)], osems.at[p]
                ).wait()

            p_splat = lane * 0 + p
            for k in range(128):
                row = plsc.load_gather(
                    pad_v, [p_splat, lane, lane * 0 + k])
                out_v[p, k] = row
            pltpu.async_copy(out_v.at[p],
                             out_hbm.at[pl.ds(c * 128, 128)],
                             osems.at[p])
            return carry

        lax.fori_loop(0, cnt, body, 0)
        pltpu.make_async_copy(out_v.at[0], out_hbm.at[pl.ds(0, 128)],
                              osems.at[0]).wait()
        pltpu.make_async_copy(out_v.at[1], out_hbm.at[pl.ds(0, 128)],
                              osems.at[1]).wait()

    return conv_kernel


@functools.lru_cache(maxsize=None)
def _make_kernel(nc, ns):
    nw = nc * ns
    bpw = _NB // nw
    ppw = _N // nw
    mesh = plsc.VectorSubcoreMesh(core_axis_name="c", subcore_axis_name="s")

    @functools.partial(
        pl.kernel,
        mesh=mesh,
        compiler_params=pltpu.CompilerParams(
            needs_layout_passes=False, use_tc_tiling_on_sc=False),
        out_type=jax.ShapeDtypeStruct((_N, _D), jnp.float32),
        scratch_types=[
            pltpu.VMEM((2, 8, _PB), jnp.int32),
            pltpu.VMEM((2, _RPB, _D), jnp.float32),
            pltpu.VMEM((ppw,), jnp.float32),
            pltpu.VMEM((ppw,), jnp.float32),
            pltpu.VMEM((ppw,), jnp.float32),
            pltpu.VMEM((_RPB,), jnp.float32),
            pltpu.VMEM((_PB, _D), jnp.float32),
            pltpu.VMEM((_L,), jnp.float32),
            pltpu.SemaphoreType.DMA((2,)),
        ],
    )
    def spc_kernel(x0_hbm, x1_hbm, x2_hbm, cidx_hbm, feat_hbm, res_hbm,
                   out_hbm, idx_v, rows_v, x0_v, x1_v, x2_v, coeff_v, out_v,
                   res_v, sems):
        wid = lax.axis_index("s") * nc + lax.axis_index("c")
        base_blk = wid * bpw
        base_pt = wid * ppw
        pltpu.sync_copy(res_hbm, res_v)
        pltpu.sync_copy(x0_hbm.at[pl.ds(pl.multiple_of(base_pt, ppw), ppw)],
                        x0_v)
        pltpu.sync_copy(x1_hbm.at[pl.ds(pl.multiple_of(base_pt, ppw), ppw)],
                        x1_v)
        pltpu.sync_copy(x2_hbm.at[pl.ds(pl.multiple_of(base_pt, ppw), ppw)],
                        x2_v)
        res = res_v[...]
        lane = lax.iota(jnp.int32, _L)

        def issue(blk, q):
            pltpu.sync_copy(cidx_hbm.at[blk], idx_v.at[q])
            for j in range(8):
                pltpu.async_copy(
                    feat_hbm.at[idx_v.at[q, j]],
                    rows_v.at[q, pl.ds(j * _PB, _PB)],
                    sems.at[q],
                )

        issue(base_blk, 0)

        def block_body(b, carry):
            p = jnp.bitwise_and(b, 1)
            q = 1 - p

            @pl.when(b < bpw - 1)
            def _():
                issue(base_blk + b + 1, q)

            pltpu.make_async_copy(
                feat_hbm.at[pl.ds(0, _RPB)], rows_v.at[p], sems.at[p]
            ).wait()

            s_in_w = b * _PB

            def grp_body(g, carry2):
                s = g * _L
                f0 = x0_v[pl.ds(s_in_w + s, _L)] * res
                f1 = x1_v[pl.ds(s_in_w + s, _L)] * res
                f2 = x2_v[pl.ds(s_in_w + s, _L)] * res
                f0 = f0 - f0.astype(jnp.int32).astype(jnp.float32)
                f1 = f1 - f1.astype(jnp.int32).astype(jnp.float32)
                f2 = f2 - f2.astype(jnp.int32).astype(jnp.float32)
                a0 = 1.0 - f0
                a1 = 1.0 - f1
                a2 = 1.0 - f2
                p00 = a0 * a1
                p01 = a0 * f1
                p10 = f0 * a1
                p11 = f0 * f1
                cs = (p00 * a2, p00 * f2, p01 * a2, p01 * f2,
                      p10 * a2, p10 * f2, p11 * a2, p11 * f2)
                sbase = lane * 8 + s * 8
                for j in range(8):
                    plsc.store_scatter(coeff_v, [sbase + j], cs[j])
                return carry2

            lax.fori_loop(0, _PB // _L, grp_body, 0)

            def pair_body(m, carry2):
                k = m * 2
                r = m * 16
                cv = coeff_v[pl.ds(r, _L)]
                acc0 = cv[0] * rows_v[p, k]
                acc1 = cv[8] * rows_v[p, k + 1]
                for j in range(1, 8):
                    acc0 = acc0 + cv[j] * rows_v[p, j * _PB + k]
                    acc1 = acc1 + cv[8 + j] * rows_v[p, j * _PB + k + 1]
                out_v[k] = acc0
                out_v[k + 1] = acc1
                return carry2

            lax.fori_loop(0, _PB // 2, pair_body, 0)
            pbase = pl.multiple_of((base_blk + b) * _PB, _PB)
            pltpu.sync_copy(out_v, out_hbm.at[pl.ds(pbase, _PB)])
            return carry

        lax.fori_loop(0, bpw, block_body, 0)

    return spc_kernel


def kernel(x, corner_idx, features, lod):
    res = (jnp.asarray(2, jnp.int32) ** (lod + _BASE_LOD)).astype(jnp.float32)
    res_vec = jnp.full((_L,), 1.0, jnp.float32) * res
    x0 = x[:, 0]
    x1 = x[:, 1]
    x2 = x[:, 2]
    cidx_view = corner_idx.T.reshape(8, _NB, _PB).transpose(1, 0, 2)
    fview = (features.T.reshape(2, 8, _V // 128, 128)
             .transpose(0, 2, 1, 3).reshape(2, _V // 16, 128))
    info = plsc.get_sparse_core_info()
    conv = _make_convert(info.num_cores, info.num_subcores)
    feat_rm = conv(fview)
    k = _make_kernel(info.num_cores, info.num_subcores)
    return k(x0, x1, x2, cidx_view, feat_rm, res_vec)


# speedup vs baseline: 2.4155x; 1.0305x over previous
import functools

import jax
import jax.numpy as jnp
from jax import lax
from jax.experimental import pallas as pl
from jax.experimental.pallas import tpu as pltpu
from jax.experimental.pallas import tpu_sc as plsc

_BASE_LOD = 9
_N = 262144
_V = 2000000
_D = 16
_L = 16

_PB = 128
_NB = _N // _PB
_RPB = _PB * 8


@functools.lru_cache(maxsize=None)
def _make_convert(nc, ns):
    nw = nc * ns
    nchunk = _V // 128
    per_w = (nchunk + nw - 1) // nw
    mesh = plsc.VectorSubcoreMesh(core_axis_name="c", subcore_axis_name="s")

    @functools.partial(
        pl.kernel,
        mesh=mesh,
        compiler_params=pltpu.CompilerParams(
            needs_layout_passes=False, use_tc_tiling_on_sc=False),
        out_type=jax.ShapeDtypeStruct((_V, _D), jnp.float32),
        scratch_types=[
            pltpu.VMEM((8, _D, 129), jnp.float32),
            pltpu.VMEM((8, 128, _D), jnp.float32),
            pltpu.SemaphoreType.DMA((8,)),
            pltpu.SemaphoreType.DMA((8,)),
        ],
    )
    def conv_kernel(fv_hbm, out_hbm, pad_v, out_v, isems, osems):
        wid = lax.axis_index("s") * nc + lax.axis_index("c")
        start = wid * per_w
        cnt = jnp.minimum(per_w, nchunk - start)
        lane = lax.iota(jnp.int32, _L)
        depth = 8

        def issue(c, q):
            m = c * 8
            pltpu.async_copy(fv_hbm.at[0, pl.ds(m, 8)],
                             pad_v.at[q, pl.ds(0, 8), pl.ds(0, 128)],
                             isems.at[q])
            pltpu.async_copy(fv_hbm.at[1, pl.ds(m, 8)],
                             pad_v.at[q, pl.ds(8, 8), pl.ds(0, 128)],
                             isems.at[q])

        for q0 in range(depth - 1):
            @pl.when(q0 < cnt)
            def _():
                issue(start + q0, q0)

        def body(i, carry):
            p = jnp.bitwise_and(i, depth - 1)
            c = start + i

            @pl.when(i + depth - 1 < cnt)
            def _():
                issue(c + depth - 1,
                      jnp.bitwise_and(i + depth - 1, depth - 1))

            pltpu.make_async_copy(
                fv_hbm.at[0, pl.ds(0, 16)],
                pad_v.at[p, pl.ds(0, 16), pl.ds(0, 128)],
                isems.at[p],
            ).wait()

            @pl.when(i >= depth)
            def _():
                pltpu.make_async_copy(
                    out_v.at[p], out_hbm.at[pl.ds(0, 128)], osems.at[p]
                ).wait()

            p_splat = lane * 0 + p
            for k in range(128):
                row = plsc.load_gather(
                    pad_v, [p_splat, lane, lane * 0 + k])
                out_v[p, k] = row
            pltpu.async_copy(out_v.at[p],
                             out_hbm.at[pl.ds(c * 128, 128)],
                             osems.at[p])
            return carry

        lax.fori_loop(0, cnt, body, 0)
        for p0 in range(depth):
            @pl.when(p0 < cnt)
            def _():
                pltpu.make_async_copy(
                    out_v.at[p0], out_hbm.at[pl.ds(0, 128)], osems.at[p0]
                ).wait()

    return conv_kernel


@functools.lru_cache(maxsize=None)
def _make_kernel(nc, ns):
    nw = nc * ns
    bpw = _NB // nw
    ppw = _N // nw
    mesh = plsc.VectorSubcoreMesh(core_axis_name="c", subcore_axis_name="s")

    @functools.partial(
        pl.kernel,
        mesh=mesh,
        compiler_params=pltpu.CompilerParams(
            needs_layout_passes=False, use_tc_tiling_on_sc=False),
        out_type=jax.ShapeDtypeStruct((_N, _D), jnp.float32),
        scratch_types=[
            pltpu.VMEM((2, 8, _PB), jnp.int32),
            pltpu.VMEM((2, _RPB, _D), jnp.float32),
            pltpu.VMEM((ppw,), jnp.float32),
            pltpu.VMEM((ppw,), jnp.float32),
            pltpu.VMEM((ppw,), jnp.float32),
            pltpu.VMEM((_RPB,), jnp.float32),
            pltpu.VMEM((_PB, _D), jnp.float32),
            pltpu.VMEM((_L,), jnp.float32),
            pltpu.SemaphoreType.DMA((2,)),
        ],
    )
    def spc_kernel(x0_hbm, x1_hbm, x2_hbm, cidx_hbm, feat_hbm, res_hbm,
                   out_hbm, idx_v, rows_v, x0_v, x1_v, x2_v, coeff_v, out_v,
                   res_v, sems):
        wid = lax.axis_index("s") * nc + lax.axis_index("c")
        base_blk = wid * bpw
        base_pt = wid * ppw
        pltpu.sync_copy(res_hbm, res_v)
        pltpu.sync_copy(x0_hbm.at[pl.ds(pl.multiple_of(base_pt, ppw), ppw)],
                        x0_v)
        pltpu.sync_copy(x1_hbm.at[pl.ds(pl.multiple_of(base_pt, ppw), ppw)],
                        x1_v)
        pltpu.sync_copy(x2_hbm.at[pl.ds(pl.multiple_of(base_pt, ppw), ppw)],
                        x2_v)
        res = res_v[...]
        lane = lax.iota(jnp.int32, _L)

        def issue(blk, q):
            pltpu.sync_copy(cidx_hbm.at[blk], idx_v.at[q])
            for j in range(8):
                pltpu.async_copy(
                    feat_hbm.at[idx_v.at[q, j]],
                    rows_v.at[q, pl.ds(j * _PB, _PB)],
                    sems.at[q],
                )

        issue(base_blk, 0)

        def block_body(b, carry):
            p = jnp.bitwise_and(b, 1)
            q = 1 - p

            @pl.when(b < bpw - 1)
            def _():
                issue(base_blk + b + 1, q)

            pltpu.make_async_copy(
                feat_hbm.at[pl.ds(0, _RPB)], rows_v.at[p], sems.at[p]
            ).wait()

            s_in_w = b * _PB

            def grp_body(g, carry2):
                s = g * _L
                f0 = x0_v[pl.ds(s_in_w + s, _L)] * res
                f1 = x1_v[pl.ds(s_in_w + s, _L)] * res
                f2 = x2_v[pl.ds(s_in_w + s, _L)] * res
                f0 = f0 - f0.astype(jnp.int32).astype(jnp.float32)
                f1 = f1 - f1.astype(jnp.int32).astype(jnp.float32)
                f2 = f2 - f2.astype(jnp.int32).astype(jnp.float32)
                a0 = 1.0 - f0
                a1 = 1.0 - f1
                a2 = 1.0 - f2
                p00 = a0 * a1
                p01 = a0 * f1
                p10 = f0 * a1
                p11 = f0 * f1
                cs = (p00 * a2, p00 * f2, p01 * a2, p01 * f2,
                      p10 * a2, p10 * f2, p11 * a2, p11 * f2)
                sbase = lane * 8 + s * 8
                for j in range(8):
                    plsc.store_scatter(coeff_v, [sbase + j], cs[j])
                return carry2

            lax.fori_loop(0, _PB // _L, grp_body, 0)

            def pair_body(m, carry2):
                k = m * 2
                r = m * 16
                cv = coeff_v[pl.ds(r, _L)]
                acc0 = cv[0] * rows_v[p, k]
                acc1 = cv[8] * rows_v[p, k + 1]
                for j in range(1, 8):
                    acc0 = acc0 + cv[j] * rows_v[p, j * _PB + k]
                    acc1 = acc1 + cv[8 + j] * rows_v[p, j * _PB + k + 1]
                out_v[k] = acc0
                out_v[k + 1] = acc1
                return carry2

            lax.fori_loop(0, _PB // 2, pair_body, 0)
            pbase = pl.multiple_of((base_blk + b) * _PB, _PB)
            pltpu.sync_copy(out_v, out_hbm.at[pl.ds(pbase, _PB)])
            return carry

        lax.fori_loop(0, bpw, block_body, 0)

    return spc_kernel


def kernel(x, corner_idx, features, lod):
    res = (jnp.asarray(2, jnp.int32) ** (lod + _BASE_LOD)).astype(jnp.float32)
    res_vec = jnp.full((_L,), 1.0, jnp.float32) * res
    x0 = x[:, 0]
    x1 = x[:, 1]
    x2 = x[:, 2]
    cidx_view = corner_idx.T.reshape(8, _NB, _PB).transpose(1, 0, 2)
    fview = (features.T.reshape(2, 8, _V // 128, 128)
             .transpose(0, 2, 1, 3).reshape(2, _V // 16, 128))
    info = plsc.get_sparse_core_info()
    conv = _make_convert(info.num_cores, info.num_subcores)
    feat_rm = conv(fview)
    k = _make_kernel(info.num_cores, info.num_subcores)
    return k(x0, x1, x2, cidx_view, feat_rm, res_vec)


# speedup vs baseline: 3.6540x; 1.5128x over previous
import functools

import jax
import jax.numpy as jnp
from jax import lax
from jax.experimental import pallas as pl
from jax.experimental.pallas import tpu as pltpu
from jax.experimental.pallas import tpu_sc as plsc

_BASE_LOD = 9
_N = 262144
_V = 2000000
_D = 16
_L = 16

_PB = 128
_NB = _N // _PB
_RPB = _PB * 8


@functools.lru_cache(maxsize=None)
def _make_convert(nc, ns):
    nw = nc * ns
    nchunk = _V // 128
    per_w = (nchunk + nw - 1) // nw
    mesh = plsc.VectorSubcoreMesh(core_axis_name="c", subcore_axis_name="s")

    @functools.partial(
        pl.kernel,
        mesh=mesh,
        compiler_params=pltpu.CompilerParams(
            needs_layout_passes=False, use_tc_tiling_on_sc=False),
        out_type=jax.ShapeDtypeStruct((_V, _D), jnp.float32),
        scratch_types=[
            pltpu.VMEM((8, _D, 128), jnp.float32),
            pltpu.VMEM((_D, 129), jnp.float32),
            pltpu.VMEM((8, 128, _D), jnp.float32),
            pltpu.SemaphoreType.DMA((8,)),
            pltpu.SemaphoreType.DMA((8,)),
        ],
    )
    def conv_kernel(fv_hbm, out_hbm, in_v, pad_v, out_v, isems, osems):
        wid = lax.axis_index("s") * nc + lax.axis_index("c")
        start = wid * per_w
        cnt = jnp.minimum(per_w, nchunk - start)
        lane = lax.iota(jnp.int32, _L)
        depth = 8

        def issue(c, q):
            m = c * 8
            pltpu.async_copy(fv_hbm.at[0, pl.ds(m, 8)],
                             in_v.at[q, pl.ds(0, 8)],
                             isems.at[q])
            pltpu.async_copy(fv_hbm.at[1, pl.ds(m, 8)],
                             in_v.at[q, pl.ds(8, 8)],
                             isems.at[q])

        for q0 in range(depth - 1):
            @pl.when(q0 < cnt)
            def _():
                issue(start + q0, q0)

        def body(i, carry):
            p = jnp.bitwise_and(i, depth - 1)
            c = start + i

            @pl.when(i + depth - 1 < cnt)
            def _():
                issue(c + depth - 1,
                      jnp.bitwise_and(i + depth - 1, depth - 1))

            pltpu.make_async_copy(
                fv_hbm.at[0, pl.ds(0, 16)],
                in_v.at[p],
                isems.at[p],
            ).wait()

            @pl.when(i >= depth)
            def _():
                pltpu.make_async_copy(
                    out_v.at[p], out_hbm.at[pl.ds(0, 128)], osems.at[p]
                ).wait()

            @plsc.parallel_loop(0, _D, unroll=4)
            def _(d):
                for s in range(8):
                    pad_v[d, pl.ds(s * _L, _L)] = in_v[p, d, pl.ds(s * _L, _L)]

            @plsc.parallel_loop(0, 128, unroll=8)
            def _(k):
                row = plsc.load_gather(pad_v, [lane, lane * 0 + k])
                out_v[p, k] = row
            pltpu.async_copy(out_v.at[p],
                             out_hbm.at[pl.ds(c * 128, 128)],
                             osems.at[p])
            return carry

        lax.fori_loop(0, cnt, body, 0)
        for p0 in range(depth):
            @pl.when(p0 < cnt)
            def _():
                pltpu.make_async_copy(
                    out_v.at[p0], out_hbm.at[pl.ds(0, 128)], osems.at[p0]
                ).wait()

    return conv_kernel


@functools.lru_cache(maxsize=None)
def _make_kernel(nc, ns):
    nw = nc * ns
    bpw = _NB // nw
    ppw = _N // nw
    mesh = plsc.VectorSubcoreMesh(core_axis_name="c", subcore_axis_name="s")

    @functools.partial(
        pl.kernel,
        mesh=mesh,
        compiler_params=pltpu.CompilerParams(
            needs_layout_passes=False, use_tc_tiling_on_sc=False),
        out_type=jax.ShapeDtypeStruct((_N, _D), jnp.float32),
        scratch_types=[
            pltpu.VMEM((2, 8, _PB), jnp.int32),
            pltpu.VMEM((2, _RPB, _D), jnp.float32),
            pltpu.VMEM((ppw,), jnp.float32),
            pltpu.VMEM((ppw,), jnp.float32),
            pltpu.VMEM((ppw,), jnp.float32),
            pltpu.VMEM((_RPB,), jnp.float32),
            pltpu.VMEM((_PB, _D), jnp.float32),
            pltpu.VMEM((_L,), jnp.float32),
            pltpu.SemaphoreType.DMA((2,)),
        ],
    )
    def spc_kernel(x0_hbm, x1_hbm, x2_hbm, cidx_hbm, feat_hbm, res_hbm,
                   out_hbm, idx_v, rows_v, x0_v, x1_v, x2_v, coeff_v, out_v,
                   res_v, sems):
        wid = lax.axis_index("s") * nc + lax.axis_index("c")
        base_blk = wid * bpw
        base_pt = wid * ppw
        pltpu.sync_copy(res_hbm, res_v)
        pltpu.sync_copy(x0_hbm.at[pl.ds(pl.multiple_of(base_pt, ppw), ppw)],
                        x0_v)
        pltpu.sync_copy(x1_hbm.at[pl.ds(pl.multiple_of(base_pt, ppw), ppw)],
                        x1_v)
        pltpu.sync_copy(x2_hbm.at[pl.ds(pl.multiple_of(base_pt, ppw), ppw)],
                        x2_v)
        res = res_v[...]
        lane = lax.iota(jnp.int32, _L)

        def issue(blk, q):
            pltpu.sync_copy(cidx_hbm.at[blk], idx_v.at[q])
            for j in range(8):
                pltpu.async_copy(
                    feat_hbm.at[idx_v.at[q, j]],
                    rows_v.at[q, pl.ds(j * _PB, _PB)],
                    sems.at[q],
                )

        issue(base_blk, 0)

        def block_body(b, carry):
            p = jnp.bitwise_and(b, 1)
            q = 1 - p

            @pl.when(b < bpw - 1)
            def _():
                issue(base_blk + b + 1, q)

            pltpu.make_async_copy(
                feat_hbm.at[pl.ds(0, _RPB)], rows_v.at[p], sems.at[p]
            ).wait()

            s_in_w = b * _PB

            def grp_body(g, carry2):
                s = g * _L
                f0 = x0_v[pl.ds(s_in_w + s, _L)] * res
                f1 = x1_v[pl.ds(s_in_w + s, _L)] * res
                f2 = x2_v[pl.ds(s_in_w + s, _L)] * res
                f0 = f0 - f0.astype(jnp.int32).astype(jnp.float32)
                f1 = f1 - f1.astype(jnp.int32).astype(jnp.float32)
                f2 = f2 - f2.astype(jnp.int32).astype(jnp.float32)
                a0 = 1.0 - f0
                a1 = 1.0 - f1
                a2 = 1.0 - f2
                p00 = a0 * a1
                p01 = a0 * f1
                p10 = f0 * a1
                p11 = f0 * f1
                cs = (p00 * a2, p00 * f2, p01 * a2, p01 * f2,
                      p10 * a2, p10 * f2, p11 * a2, p11 * f2)
                sbase = lane * 8 + s * 8
                for j in range(8):
                    plsc.store_scatter(coeff_v, [sbase + j], cs[j])
                return carry2

            lax.fori_loop(0, _PB // _L, grp_body, 0)

            def pair_body(m, carry2):
                k = m * 2
                r = m * 16
                cv = coeff_v[pl.ds(r, _L)]
                acc0 = cv[0] * rows_v[p, k]
                acc1 = cv[8] * rows_v[p, k + 1]
                for j in range(1, 8):
                    acc0 = acc0 + cv[j] * rows_v[p, j * _PB + k]
                    acc1 = acc1 + cv[8 + j] * rows_v[p, j * _PB + k + 1]
                out_v[k] = acc0
                out_v[k + 1] = acc1
                return carry2

            lax.fori_loop(0, _PB // 2, pair_body, 0)
            pbase = pl.multiple_of((base_blk + b) * _PB, _PB)
            pltpu.sync_copy(out_v, out_hbm.at[pl.ds(pbase, _PB)])
            return carry

        lax.fori_loop(0, bpw, block_body, 0)

    return spc_kernel


def kernel(x, corner_idx, features, lod):
    res = (jnp.asarray(2, jnp.int32) ** (lod + _BASE_LOD)).astype(jnp.float32)
    res_vec = jnp.full((_L,), 1.0, jnp.float32) * res
    x0 = x[:, 0]
    x1 = x[:, 1]
    x2 = x[:, 2]
    cidx_view = corner_idx.T.reshape(8, _NB, _PB).transpose(1, 0, 2)
    fview = (features.T.reshape(2, 8, _V // 128, 128)
             .transpose(0, 2, 1, 3).reshape(2, _V // 16, 128))
    info = plsc.get_sparse_core_info()
    conv = _make_convert(info.num_cores, info.num_subcores)
    feat_rm = conv(fview)
    k = _make_kernel(info.num_cores, info.num_subcores)
    return k(x0, x1, x2, cidx_view, feat_rm, res_vec)


# speedup vs baseline: 5.3374x; 1.4607x over previous
import functools

import jax
import jax.numpy as jnp
from jax import lax
from jax.experimental import pallas as pl
from jax.experimental.pallas import tpu as pltpu
from jax.experimental.pallas import tpu_sc as plsc

_BASE_LOD = 9
_N = 262144
_V = 2000000
_D = 16
_L = 16

_PB = 128
_NB = _N // _PB
_RPB = _PB * 8


@functools.lru_cache(maxsize=None)
def _make_convert(nc, ns):
    nw = nc * ns
    nchunk = _V // 128
    per_w = (nchunk + nw - 1) // nw
    mesh = plsc.VectorSubcoreMesh(core_axis_name="c", subcore_axis_name="s")

    @functools.partial(
        pl.kernel,
        mesh=mesh,
        compiler_params=pltpu.CompilerParams(
            needs_layout_passes=False, use_tc_tiling_on_sc=False),
        out_type=jax.ShapeDtypeStruct((_V, _D), jnp.float32),
        scratch_types=[
            pltpu.VMEM((8, _D, 128), jnp.float32),
            pltpu.VMEM((_D, 129), jnp.float32),
            pltpu.VMEM((8, 128, _D), jnp.float32),
            pltpu.SemaphoreType.DMA((8,)),
            pltpu.SemaphoreType.DMA((8,)),
        ],
    )
    def conv_kernel(fv_hbm, out_hbm, in_v, pad_v, out_v, isems, osems):
        wid = lax.axis_index("s") * nc + lax.axis_index("c")
        start = wid * per_w
        cnt = jnp.minimum(per_w, nchunk - start)
        lane = lax.iota(jnp.int32, _L)
        depth = 8

        def issue(c, q):
            m = c * 8
            pltpu.async_copy(fv_hbm.at[0, pl.ds(m, 8)],
                             in_v.at[q, pl.ds(0, 8)],
                             isems.at[q])
            pltpu.async_copy(fv_hbm.at[1, pl.ds(m, 8)],
                             in_v.at[q, pl.ds(8, 8)],
                             isems.at[q])

        for q0 in range(depth - 1):
            @pl.when(q0 < cnt)
            def _():
                issue(start + q0, q0)

        def body(i, carry):
            p = jnp.bitwise_and(i, depth - 1)
            c = start + i

            @pl.when(i + depth - 1 < cnt)
            def _():
                issue(c + depth - 1,
                      jnp.bitwise_and(i + depth - 1, depth - 1))

            pltpu.make_async_copy(
                fv_hbm.at[0, pl.ds(0, 16)],
                in_v.at[p],
                isems.at[p],
            ).wait()

            @pl.when(i >= depth)
            def _():
                pltpu.make_async_copy(
                    out_v.at[p], out_hbm.at[pl.ds(0, 128)], osems.at[p]
                ).wait()

            @plsc.parallel_loop(0, _D, unroll=4)
            def _(d):
                for s in range(8):
                    pad_v[d, pl.ds(s * _L, _L)] = in_v[p, d, pl.ds(s * _L, _L)]

            @plsc.parallel_loop(0, 128, unroll=8)
            def _(k):
                row = plsc.load_gather(pad_v, [lane, lane * 0 + k])
                out_v[p, k] = row
            pltpu.async_copy(out_v.at[p],
                             out_hbm.at[pl.ds(c * 128, 128)],
                             osems.at[p])
            return carry

        lax.fori_loop(0, cnt, body, 0)
        for p0 in range(depth):
            @pl.when(p0 < cnt)
            def _():
                pltpu.make_async_copy(
                    out_v.at[p0], out_hbm.at[pl.ds(0, 128)], osems.at[p0]
                ).wait()

    return conv_kernel


@functools.lru_cache(maxsize=None)
def _make_kernel(nc, ns):
    nw = nc * ns
    bpw = _NB // nw
    ppw = _N // nw
    mesh = plsc.VectorSubcoreMesh(core_axis_name="c", subcore_axis_name="s")

    @functools.partial(
        pl.kernel,
        mesh=mesh,
        compiler_params=pltpu.CompilerParams(
            needs_layout_passes=False, use_tc_tiling_on_sc=False),
        out_type=jax.ShapeDtypeStruct((2, _NB, 8, _PB), jnp.float32),
        scratch_types=[
            pltpu.VMEM((2, 8, _PB), jnp.int32),
            pltpu.VMEM((2, _RPB, _D), jnp.float32),
            pltpu.VMEM((ppw,), jnp.float32),
            pltpu.VMEM((ppw,), jnp.float32),
            pltpu.VMEM((ppw,), jnp.float32),
            pltpu.VMEM((_RPB,), jnp.float32),
            pltpu.VMEM((_PB, 17), jnp.float32),
            pltpu.VMEM((2, _D, _PB), jnp.float32),
            pltpu.VMEM((_L,), jnp.float32),
            pltpu.SemaphoreType.DMA((2,)),
            pltpu.SemaphoreType.DMA((2,)),
        ],
    )
    def spc_kernel(x0_hbm, x1_hbm, x2_hbm, cidx_hbm, feat_hbm, res_hbm,
                   out_hbm, idx_v, rows_v, x0_v, x1_v, x2_v, coeff_v, pad_v,
                   outT_v, res_v, sems, osems):
        wid = lax.axis_index("s") * nc + lax.axis_index("c")
        base_blk = wid * bpw
        base_pt = wid * ppw
        pltpu.sync_copy(res_hbm, res_v)
        pltpu.sync_copy(x0_hbm.at[pl.ds(pl.multiple_of(base_pt, ppw), ppw)],
                        x0_v)
        pltpu.sync_copy(x1_hbm.at[pl.ds(pl.multiple_of(base_pt, ppw), ppw)],
                        x1_v)
        pltpu.sync_copy(x2_hbm.at[pl.ds(pl.multiple_of(base_pt, ppw), ppw)],
                        x2_v)
        res = res_v[...]
        lane = lax.iota(jnp.int32, _L)

        def issue(blk, q):
            pltpu.sync_copy(cidx_hbm.at[blk], idx_v.at[q])
            for j in range(8):
                pltpu.async_copy(
                    feat_hbm.at[idx_v.at[q, j]],
                    rows_v.at[q, pl.ds(j * _PB, _PB)],
                    sems.at[q],
                )

        issue(base_blk, 0)

        def block_body(b, carry):
            p = jnp.bitwise_and(b, 1)
            q = 1 - p

            @pl.when(b < bpw - 1)
            def _():
                issue(base_blk + b + 1, q)

            pltpu.make_async_copy(
                feat_hbm.at[pl.ds(0, _RPB)], rows_v.at[p], sems.at[p]
            ).wait()

            s_in_w = b * _PB

            @plsc.parallel_loop(0, _PB // _L, unroll=2)
            def grp_body(g):
                s = g * _L
                f0 = x0_v[pl.ds(s_in_w + s, _L)] * res
                f1 = x1_v[pl.ds(s_in_w + s, _L)] * res
                f2 = x2_v[pl.ds(s_in_w + s, _L)] * res
                f0 = f0 - f0.astype(jnp.int32).astype(jnp.float32)
                f1 = f1 - f1.astype(jnp.int32).astype(jnp.float32)
                f2 = f2 - f2.astype(jnp.int32).astype(jnp.float32)
                a0 = 1.0 - f0
                a1 = 1.0 - f1
                a2 = 1.0 - f2
                p00 = a0 * a1
                p01 = a0 * f1
                p10 = f0 * a1
                p11 = f0 * f1
                cs = (p00 * a2, p00 * f2, p01 * a2, p01 * f2,
                      p10 * a2, p10 * f2, p11 * a2, p11 * f2)
                sbase = lane * 8 + s * 8
                for j in range(8):
                    plsc.store_scatter(coeff_v, [sbase + j], cs[j])

            @plsc.parallel_loop(0, _PB // 2, unroll=4)
            def pair_body(m):
                k = m * 2
                r = m * 16
                cv = coeff_v[pl.ds(r, _L)]
                acc0 = cv[0] * rows_v[p, k]
                acc1 = cv[8] * rows_v[p, k + 1]
                for j in range(1, 8):
                    acc0 = acc0 + cv[j] * rows_v[p, j * _PB + k]
                    acc1 = acc1 + cv[8 + j] * rows_v[p, j * _PB + k + 1]
                pad_v[k, pl.ds(0, _L)] = acc0
                pad_v[k + 1, pl.ds(0, _L)] = acc1

            @pl.when(b >= 2)
            def _():
                pltpu.make_async_copy(
                    outT_v.at[p], out_hbm.at[0, pl.ds(0, 2)], osems.at[p]
                ).wait()

            @plsc.parallel_loop(0, 8 * _D, unroll=8)
            def trans_body(t):
                d = lax.div(t, 8)
                s = lax.rem(t, 8)
                row = plsc.load_gather(
                    pad_v, [s * _L + lane, lane * 0 + d])
                outT_v[p, d, pl.ds(s * _L, _L)] = row

            cb_out = base_blk + b
            pltpu.async_copy(outT_v.at[p, pl.ds(0, 8)],
                             out_hbm.at[0, cb_out], osems.at[p])
            pltpu.async_copy(outT_v.at[p, pl.ds(8, 8)],
                             out_hbm.at[1, cb_out], osems.at[p])
            return carry

        lax.fori_loop(0, bpw, block_body, 0)
        for p0 in range(2):
            pltpu.make_async_copy(
                outT_v.at[p0], out_hbm.at[0, pl.ds(0, 2)], osems.at[p0]
            ).wait()

    return spc_kernel


def kernel(x, corner_idx, features, lod):
    res = (jnp.asarray(2, jnp.int32) ** (lod + _BASE_LOD)).astype(jnp.float32)
    res_vec = jnp.full((_L,), 1.0, jnp.float32) * res
    x0 = x[:, 0]
    x1 = x[:, 1]
    x2 = x[:, 2]
    cidx_view = corner_idx.T.reshape(8, _NB, _PB).transpose(1, 0, 2)
    fview = (features.T.reshape(2, 8, _V // 128, 128)
             .transpose(0, 2, 1, 3).reshape(2, _V // 16, 128))
    info = plsc.get_sparse_core_info()
    conv = _make_convert(info.num_cores, info.num_subcores)
    feat_rm = conv(fview)
    k = _make_kernel(info.num_cores, info.num_subcores)
    out_view = k(x0, x1, x2, cidx_view, feat_rm, res_vec)
    return out_view.transpose(0, 2, 1, 3).reshape(_D, _N).T


# speedup vs baseline: 5.3427x; 1.0010x over previous
import functools

import jax
import jax.numpy as jnp
from jax import lax
from jax.experimental import pallas as pl
from jax.experimental.pallas import tpu as pltpu
from jax.experimental.pallas import tpu_sc as plsc

_BASE_LOD = 9
_N = 262144
_V = 2000000
_D = 16
_L = 16

_PB = 128
_NB = _N // _PB
_RPB = _PB * 8


@functools.lru_cache(maxsize=None)
def _make_convert(nc, ns):
    nw = nc * ns
    nchunk = _V // 128
    per_w = (nchunk + nw - 1) // nw
    mesh = plsc.VectorSubcoreMesh(core_axis_name="c", subcore_axis_name="s")

    @functools.partial(
        pl.kernel,
        mesh=mesh,
        compiler_params=pltpu.CompilerParams(
            needs_layout_passes=False, use_tc_tiling_on_sc=False),
        out_type=jax.ShapeDtypeStruct((_V, _D), jnp.float32),
        scratch_types=[
            pltpu.VMEM((8, _D, 128), jnp.float32),
            pltpu.VMEM((_D, 129), jnp.float32),
            pltpu.VMEM((8, 128, _D), jnp.float32),
            pltpu.SemaphoreType.DMA((8,)),
            pltpu.SemaphoreType.DMA((8,)),
        ],
    )
    def conv_kernel(fv_hbm, out_hbm, in_v, pad_v, out_v, isems, osems):
        wid = lax.axis_index("s") * nc + lax.axis_index("c")
        start = wid * per_w
        cnt = jnp.minimum(per_w, nchunk - start)
        lane = lax.iota(jnp.int32, _L)
        depth = 8

        def issue(c, q):
            m = c * 8
            pltpu.async_copy(fv_hbm.at[0, pl.ds(m, 8)],
                             in_v.at[q, pl.ds(0, 8)],
                             isems.at[q])
            pltpu.async_copy(fv_hbm.at[1, pl.ds(m, 8)],
                             in_v.at[q, pl.ds(8, 8)],
                             isems.at[q])

        for q0 in range(depth - 1):
            @pl.when(q0 < cnt)
            def _():
                issue(start + q0, q0)

        def body(i, carry):
            p = jnp.bitwise_and(i, depth - 1)
            c = start + i

            @pl.when(i + depth - 1 < cnt)
            def _():
                issue(c + depth - 1,
                      jnp.bitwise_and(i + depth - 1, depth - 1))

            pltpu.make_async_copy(
                fv_hbm.at[0, pl.ds(0, 16)],
                in_v.at[p],
                isems.at[p],
            ).wait()

            @pl.when(i >= depth)
            def _():
                pltpu.make_async_copy(
                    out_v.at[p], out_hbm.at[pl.ds(0, 128)], osems.at[p]
                ).wait()

            @plsc.parallel_loop(0, _D, unroll=4)
            def _(d):
                for s in range(8):
                    pad_v[d, pl.ds(s * _L, _L)] = in_v[p, d, pl.ds(s * _L, _L)]

            @plsc.parallel_loop(0, 128, unroll=8)
            def _(k):
                row = plsc.load_gather(pad_v, [lane, lane * 0 + k])
                out_v[p, k] = row
            pltpu.async_copy(out_v.at[p],
                             out_hbm.at[pl.ds(c * 128, 128)],
                             osems.at[p])
            return carry

        lax.fori_loop(0, cnt, body, 0)
        for p0 in range(depth):
            @pl.when(p0 < cnt)
            def _():
                pltpu.make_async_copy(
                    out_v.at[p0], out_hbm.at[pl.ds(0, 128)], osems.at[p0]
                ).wait()

    return conv_kernel


@functools.lru_cache(maxsize=None)
def _make_kernel(nc, ns):
    nw = nc * ns
    bpw = _NB // nw
    ppw = _N // nw
    mesh = plsc.VectorSubcoreMesh(core_axis_name="c", subcore_axis_name="s")

    @functools.partial(
        pl.kernel,
        mesh=mesh,
        compiler_params=pltpu.CompilerParams(
            needs_layout_passes=False, use_tc_tiling_on_sc=False),
        out_type=jax.ShapeDtypeStruct((2, _NB, 8, _PB), jnp.float32),
        scratch_types=[
            pltpu.VMEM((2, 8, _PB), jnp.int32),
            pltpu.VMEM((2, _RPB, _D), jnp.float32),
            pltpu.VMEM((ppw,), jnp.float32),
            pltpu.VMEM((ppw,), jnp.float32),
            pltpu.VMEM((ppw,), jnp.float32),
            pltpu.VMEM((_RPB,), jnp.float32),
            pltpu.VMEM((_PB, 17), jnp.float32),
            pltpu.VMEM((2, _D, _PB), jnp.float32),
            pltpu.VMEM((_L,), jnp.float32),
            pltpu.SemaphoreType.DMA((2,)),
            pltpu.SemaphoreType.DMA((2,)),
        ],
    )
    def spc_kernel(x0_hbm, x1_hbm, x2_hbm, cidx_hbm, feat_hbm, res_hbm,
                   out_hbm, idx_v, rows_v, x0_v, x1_v, x2_v, coeff_v, pad_v,
                   outT_v, res_v, sems, osems):
        wid = lax.axis_index("s") * nc + lax.axis_index("c")
        base_blk = wid * bpw
        base_pt = wid * ppw
        pltpu.sync_copy(res_hbm, res_v)
        pltpu.sync_copy(x0_hbm.at[pl.ds(pl.multiple_of(base_pt, ppw), ppw)],
                        x0_v)
        pltpu.sync_copy(x1_hbm.at[pl.ds(pl.multiple_of(base_pt, ppw), ppw)],
                        x1_v)
        pltpu.sync_copy(x2_hbm.at[pl.ds(pl.multiple_of(base_pt, ppw), ppw)],
                        x2_v)
        res = res_v[...]
        lane = lax.iota(jnp.int32, _L)

        def issue(blk, q):
            pltpu.sync_copy(cidx_hbm.at[blk], idx_v.at[q])
            for j in range(8):
                pltpu.async_copy(
                    feat_hbm.at[idx_v.at[q, j]],
                    rows_v.at[q, pl.ds(j * _PB, _PB)],
                    sems.at[q],
                )

        issue(base_blk, 0)

        def block_body(b, carry):
            p = jnp.bitwise_and(b, 1)
            q = 1 - p

            @pl.when(b < bpw - 1)
            def _():
                issue(base_blk + b + 1, q)

            pltpu.make_async_copy(
                feat_hbm.at[pl.ds(0, _RPB)], rows_v.at[p], sems.at[p]
            ).wait()

            s_in_w = b * _PB

            @plsc.parallel_loop(0, _PB // _L, unroll=4)
            def grp_body(g):
                s = g * _L
                f0 = x0_v[pl.ds(s_in_w + s, _L)] * res
                f1 = x1_v[pl.ds(s_in_w + s, _L)] * res
                f2 = x2_v[pl.ds(s_in_w + s, _L)] * res
                f0 = f0 - f0.astype(jnp.int32).astype(jnp.float32)
                f1 = f1 - f1.astype(jnp.int32).astype(jnp.float32)
                f2 = f2 - f2.astype(jnp.int32).astype(jnp.float32)
                a0 = 1.0 - f0
                a1 = 1.0 - f1
                a2 = 1.0 - f2
                p00 = a0 * a1
                p01 = a0 * f1
                p10 = f0 * a1
                p11 = f0 * f1
                cs = (p00 * a2, p00 * f2, p01 * a2, p01 * f2,
                      p10 * a2, p10 * f2, p11 * a2, p11 * f2)
                sbase = lane * 8 + s * 8
                for j in range(8):
                    plsc.store_scatter(coeff_v, [sbase + j], cs[j])

            @plsc.parallel_loop(0, _PB // 2, unroll=8)
            def pair_body(m):
                k = m * 2
                r = m * 16
                cv = coeff_v[pl.ds(r, _L)]
                acc0 = cv[0] * rows_v[p, k]
                acc1 = cv[8] * rows_v[p, k + 1]
                for j in range(1, 8):
                    acc0 = acc0 + cv[j] * rows_v[p, j * _PB + k]
                    acc1 = acc1 + cv[8 + j] * rows_v[p, j * _PB + k + 1]
                pad_v[k, pl.ds(0, _L)] = acc0
                pad_v[k + 1, pl.ds(0, _L)] = acc1

            @pl.when(b >= 2)
            def _():
                pltpu.make_async_copy(
                    outT_v.at[p], out_hbm.at[0, pl.ds(0, 2)], osems.at[p]
                ).wait()

            @plsc.parallel_loop(0, 8 * _D, unroll=8)
            def trans_body(t):
                d = lax.div(t, 8)
                s = lax.rem(t, 8)
                row = plsc.load_gather(
                    pad_v, [s * _L + lane, lane * 0 + d])
                outT_v[p, d, pl.ds(s * _L, _L)] = row

            cb_out = base_blk + b
            pltpu.async_copy(outT_v.at[p, pl.ds(0, 8)],
                             out_hbm.at[0, cb_out], osems.at[p])
            pltpu.async_copy(outT_v.at[p, pl.ds(8, 8)],
                             out_hbm.at[1, cb_out], osems.at[p])
            return carry

        lax.fori_loop(0, bpw, block_body, 0)
        for p0 in range(2):
            pltpu.make_async_copy(
                outT_v.at[p0], out_hbm.at[0, pl.ds(0, 2)], osems.at[p0]
            ).wait()

    return spc_kernel


def kernel(x, corner_idx, features, lod):
    res = (jnp.asarray(2, jnp.int32) ** (lod + _BASE_LOD)).astype(jnp.float32)
    res_vec = jnp.full((_L,), 1.0, jnp.float32) * res
    x0 = x[:, 0]
    x1 = x[:, 1]
    x2 = x[:, 2]
    cidx_view = corner_idx.T.reshape(8, _NB, _PB).transpose(1, 0, 2)
    fview = (features.T.reshape(2, 8, _V // 128, 128)
             .transpose(0, 2, 1, 3).reshape(2, _V // 16, 128))
    info = plsc.get_sparse_core_info()
    conv = _make_convert(info.num_cores, info.num_subcores)
    feat_rm = conv(fview)
    k = _make_kernel(info.num_cores, info.num_subcores)
    out_view = k(x0, x1, x2, cidx_view, feat_rm, res_vec)
    return out_view.transpose(0, 2, 1, 3).reshape(_D, _N).T


# speedup vs baseline: 5.5451x; 1.0379x over previous
import functools

import jax
import jax.numpy as jnp
from jax import lax
from jax.experimental import pallas as pl
from jax.experimental.pallas import tpu as pltpu
from jax.experimental.pallas import tpu_sc as plsc

_BASE_LOD = 9
_N = 262144
_V = 2000000
_D = 16
_L = 16

_PB = 128
_NB = _N // _PB
_RPB = _PB * 8


@functools.lru_cache(maxsize=None)
def _make_convert(nc, ns):
    nw = nc * ns
    nchunk = _V // 128
    per_w = (nchunk + nw - 1) // nw
    mesh = plsc.VectorSubcoreMesh(core_axis_name="c", subcore_axis_name="s")

    @functools.partial(
        pl.kernel,
        mesh=mesh,
        compiler_params=pltpu.CompilerParams(
            needs_layout_passes=False, use_tc_tiling_on_sc=False),
        out_type=jax.ShapeDtypeStruct((_V, _D), jnp.float32),
        scratch_types=[
            pltpu.VMEM((8, _D, 128), jnp.float32),
            pltpu.VMEM((_D, 129), jnp.float32),
            pltpu.VMEM((8, 128, _D), jnp.float32),
            pltpu.SemaphoreType.DMA((8,)),
            pltpu.SemaphoreType.DMA((8,)),
        ],
    )
    def conv_kernel(fv_hbm, out_hbm, in_v, pad_v, out_v, isems, osems):
        wid = lax.axis_index("s") * nc + lax.axis_index("c")
        start = wid * per_w
        cnt = jnp.minimum(per_w, nchunk - start)
        lane = lax.iota(jnp.int32, _L)
        depth = 8

        def issue(c, q):
            m = c * 8
            pltpu.async_copy(fv_hbm.at[0, pl.ds(m, 8)],
                             in_v.at[q, pl.ds(0, 8)],
                             isems.at[q])
            pltpu.async_copy(fv_hbm.at[1, pl.ds(m, 8)],
                             in_v.at[q, pl.ds(8, 8)],
                             isems.at[q])

        for q0 in range(depth - 1):
            @pl.when(q0 < cnt)
            def _():
                issue(start + q0, q0)

        def body(i, carry):
            p = jnp.bitwise_and(i, depth - 1)
            c = start + i

            @pl.when(i + depth - 1 < cnt)
            def _():
                issue(c + depth - 1,
                      jnp.bitwise_and(i + depth - 1, depth - 1))

            pltpu.make_async_copy(
                fv_hbm.at[0, pl.ds(0, 16)],
                in_v.at[p],
                isems.at[p],
            ).wait()

            @pl.when(i >= depth)
            def _():
                pltpu.make_async_copy(
                    out_v.at[p], out_hbm.at[pl.ds(0, 128)], osems.at[p]
                ).wait()

            @plsc.parallel_loop(0, _D, unroll=4)
            def _(d):
                for s in range(8):
                    pad_v[d, pl.ds(s * _L, _L)] = in_v[p, d, pl.ds(s * _L, _L)]

            @plsc.parallel_loop(0, 128, unroll=8)
            def _(k):
                row = plsc.load_gather(pad_v, [lane, lane * 0 + k])
                out_v[p, k] = row
            pltpu.async_copy(out_v.at[p],
                             out_hbm.at[pl.ds(c * 128, 128)],
                             osems.at[p])
            return carry

        lax.fori_loop(0, cnt, body, 0)
        for p0 in range(depth):
            @pl.when(p0 < cnt)
            def _():
                pltpu.make_async_copy(
                    out_v.at[p0], out_hbm.at[pl.ds(0, 128)], osems.at[p0]
                ).wait()

    return conv_kernel


@functools.lru_cache(maxsize=None)
def _make_kernel(nc, ns):
    nw = nc * ns
    bpw = _NB // nw
    ppw = _N // nw
    mesh = plsc.VectorSubcoreMesh(core_axis_name="c", subcore_axis_name="s")

    @functools.partial(
        pl.kernel,
        mesh=mesh,
        compiler_params=pltpu.CompilerParams(
            needs_layout_passes=False, use_tc_tiling_on_sc=False),
        out_type=jax.ShapeDtypeStruct((2, _NB, 8, _PB), jnp.float32),
        scratch_types=[
            pltpu.VMEM((4, 8, _PB), jnp.int32),
            pltpu.VMEM((4, _RPB, _D), jnp.float32),
            pltpu.VMEM((ppw,), jnp.float32),
            pltpu.VMEM((ppw,), jnp.float32),
            pltpu.VMEM((ppw,), jnp.float32),
            pltpu.VMEM((_RPB,), jnp.float32),
            pltpu.VMEM((_PB, 17), jnp.float32),
            pltpu.VMEM((2, _D, _PB), jnp.float32),
            pltpu.VMEM((_L,), jnp.float32),
            pltpu.SemaphoreType.DMA((4,)),
            pltpu.SemaphoreType.DMA((2,)),
        ],
    )
    def spc_kernel(x0_hbm, x1_hbm, x2_hbm, cidx_hbm, feat_hbm, res_hbm,
                   out_hbm, idx_v, rows_v, x0_v, x1_v, x2_v, coeff_v, pad_v,
                   outT_v, res_v, sems, osems):
        wid = lax.axis_index("s") * nc + lax.axis_index("c")
        base_blk = wid * bpw
        base_pt = wid * ppw
        pltpu.sync_copy(res_hbm, res_v)
        pltpu.sync_copy(x0_hbm.at[pl.ds(pl.multiple_of(base_pt, ppw), ppw)],
                        x0_v)
        pltpu.sync_copy(x1_hbm.at[pl.ds(pl.multiple_of(base_pt, ppw), ppw)],
                        x1_v)
        pltpu.sync_copy(x2_hbm.at[pl.ds(pl.multiple_of(base_pt, ppw), ppw)],
                        x2_v)
        res = res_v[...]
        lane = lax.iota(jnp.int32, _L)

        def issue(blk, q):
            pltpu.sync_copy(cidx_hbm.at[blk], idx_v.at[q])
            for j in range(8):
                pltpu.async_copy(
                    feat_hbm.at[idx_v.at[q, j]],
                    rows_v.at[q, pl.ds(j * _PB, _PB)],
                    sems.at[q],
                )

        for q0 in range(3):
            issue(base_blk + q0, q0)

        def block_body(b, carry):
            p = jnp.bitwise_and(b, 3)

            @pl.when(b < bpw - 3)
            def _():
                issue(base_blk + b + 3, jnp.bitwise_and(b + 3, 3))

            pltpu.make_async_copy(
                feat_hbm.at[pl.ds(0, _RPB)], rows_v.at[p], sems.at[p]
            ).wait()

            s_in_w = b * _PB

            @plsc.parallel_loop(0, _PB // _L, unroll=4)
            def grp_body(g):
                s = g * _L
                f0 = x0_v[pl.ds(s_in_w + s, _L)] * res
                f1 = x1_v[pl.ds(s_in_w + s, _L)] * res
                f2 = x2_v[pl.ds(s_in_w + s, _L)] * res
                f0 = f0 - f0.astype(jnp.int32).astype(jnp.float32)
                f1 = f1 - f1.astype(jnp.int32).astype(jnp.float32)
                f2 = f2 - f2.astype(jnp.int32).astype(jnp.float32)
                a0 = 1.0 - f0
                a1 = 1.0 - f1
                a2 = 1.0 - f2
                p00 = a0 * a1
                p01 = a0 * f1
                p10 = f0 * a1
                p11 = f0 * f1
                cs = (p00 * a2, p00 * f2, p01 * a2, p01 * f2,
                      p10 * a2, p10 * f2, p11 * a2, p11 * f2)
                sbase = lane * 8 + s * 8
                for j in range(8):
                    plsc.store_scatter(coeff_v, [sbase + j], cs[j])

            @plsc.parallel_loop(0, _PB // 2, unroll=8)
            def pair_body(m):
                k = m * 2
                r = m * 16
                cv = coeff_v[pl.ds(r, _L)]
                acc0 = cv[0] * rows_v[p, k]
                acc1 = cv[8] * rows_v[p, k + 1]
                for j in range(1, 8):
                    acc0 = acc0 + cv[j] * rows_v[p, j * _PB + k]
                    acc1 = acc1 + cv[8 + j] * rows_v[p, j * _PB + k + 1]
                pad_v[k, pl.ds(0, _L)] = acc0
                pad_v[k + 1, pl.ds(0, _L)] = acc1

            po = jnp.bitwise_and(b, 1)

            @pl.when(b >= 2)
            def _():
                pltpu.make_async_copy(
                    outT_v.at[po], out_hbm.at[0, pl.ds(0, 2)], osems.at[po]
                ).wait()

            @plsc.parallel_loop(0, 8 * _D, unroll=8)
            def trans_body(t):
                d = lax.div(t, 8)
                s = lax.rem(t, 8)
                row = plsc.load_gather(
                    pad_v, [s * _L + lane, lane * 0 + d])
                outT_v[po, d, pl.ds(s * _L, _L)] = row

            cb_out = base_blk + b
            pltpu.async_copy(outT_v.at[po, pl.ds(0, 8)],
                             out_hbm.at[0, cb_out], osems.at[po])
            pltpu.async_copy(outT_v.at[po, pl.ds(8, 8)],
                             out_hbm.at[1, cb_out], osems.at[po])
            return carry

        lax.fori_loop(0, bpw, block_body, 0)
        for p0 in range(2):
            pltpu.make_async_copy(
                outT_v.at[p0], out_hbm.at[0, pl.ds(0, 2)], osems.at[p0]
            ).wait()

    return spc_kernel


def kernel(x, corner_idx, features, lod):
    res = (jnp.asarray(2, jnp.int32) ** (lod + _BASE_LOD)).astype(jnp.float32)
    res_vec = jnp.full((_L,), 1.0, jnp.float32) * res
    x0 = x[:, 0]
    x1 = x[:, 1]
    x2 = x[:, 2]
    cidx_view = corner_idx.T.reshape(8, _NB, _PB).transpose(1, 0, 2)
    fview = (features.T.reshape(2, 8, _V // 128, 128)
             .transpose(0, 2, 1, 3).reshape(2, _V // 16, 128))
    info = plsc.get_sparse_core_info()
    conv = _make_convert(info.num_cores, info.num_subcores)
    feat_rm = conv(fview)
    k = _make_kernel(info.num_cores, info.num_subcores)
    out_view = k(x0, x1, x2, cidx_view, feat_rm, res_vec)
    return out_view.transpose(0, 2, 1, 3).reshape(_D, _N).T
